# baseline (speedup 1.0000x reference)
import math

import jax
import jax.numpy as jnp
from jax.experimental import pallas as pl
from jax.experimental.pallas import tpu as pltpu

_BW2 = 16.0
_EPS = 1e-30
_KC = 100352
_DA = 24
_LANES = 128


def _make_body(norm_const):
    def body(qa_ref, ta_ref, w_ref, out_ref, acc_ref, ws_ref):
        k = pl.program_id(0)
        nk = pl.num_programs(0)

        @pl.when(k == 0)
        def _init():
            acc_ref[...] = jnp.zeros_like(acc_ref)
            ws_ref[0, 0] = 0.0

        arg = jnp.dot(qa_ref[...], ta_ref[...],
                      preferred_element_type=jnp.float32)
        ws_ref[0, 0] += jnp.sum(w_ref[0])

        ex = jax.lax.exp2(arg.astype(jnp.bfloat16))
        parts = [ex[:, g * _LANES:(g + 1) * _LANES]
                 for g in range(_KC // _LANES)]
        while len(parts) > 1:
            nxt = [parts[i] + parts[i + 1]
                   for i in range(0, len(parts) - 1, 2)]
            if len(parts) % 2:
                nxt.append(parts[-1])
            parts = nxt
        acc_ref[...] += parts[0].astype(jnp.float32)

        @pl.when(k == nk - 1)
        def _epilogue():
            dens = jnp.sum(acc_ref[...], axis=1, keepdims=True)
            dens = dens * (norm_const / ws_ref[0, 0])
            out_ref[...] = jnp.log(jnp.maximum(dens, _EPS))

    return body


def kernel(X_query, X_train, sample_weight):
    n, d = X_query.shape
    K = X_train.shape[0]
    norm_const = (2.0 * math.pi * _BW2) ** (-d / 2.0)
    c2 = -1.0 / (2.0 * _BW2 * math.log(2.0))

    kpad = ((K + _KC - 1) // _KC) * _KC
    nk = kpad // _KC

    q_sq = jnp.sum(X_query * X_query, axis=1, keepdims=True)
    t_sq = jnp.sum(X_train * X_train, axis=1)
    lw = jnp.log2(jnp.maximum(sample_weight, 1e-38))

    ones_n = jnp.ones((n, 1), jnp.float32)
    qa = jnp.concatenate(
        [(-2.0 * c2) * X_query, c2 * q_sq, c2 * ones_n, ones_n], axis=1)
    qa = jnp.pad(qa, ((0, 0), (0, _DA - (d + 3)))).astype(jnp.bfloat16)

    ta = jnp.concatenate(
        [X_train.T, jnp.ones((1, K), jnp.float32), t_sq[None, :],
         lw[None, :]], axis=0)
    ta = jnp.pad(ta, ((0, _DA - (d + 3)), (0, 0)))
    ta = jnp.pad(ta, ((0, 0), (0, kpad - K)),
                 constant_values=0.0).at[d + 2, K:].set(-126.0)
    ta = ta.astype(jnp.bfloat16)

    w = jnp.pad(sample_weight, (0, kpad - K)).reshape(nk, 1, _KC)

    out = pl.pallas_call(
        _make_body(norm_const),
        grid=(nk,),
        in_specs=[
            pl.BlockSpec((n, _DA), lambda k: (0, 0)),
            pl.BlockSpec((_DA, _KC), lambda k: (0, k)),
            pl.BlockSpec((1, 1, _KC), lambda k: (k, 0, 0)),
        ],
        out_specs=pl.BlockSpec((n, 1), lambda k: (0, 0)),
        out_shape=jax.ShapeDtypeStruct((n, 1), jnp.float32),
        compiler_params=pltpu.CompilerParams(
            allow_input_fusion=[True, True, True]),
        scratch_shapes=[
            pltpu.VMEM((n, _LANES), jnp.float32),
            pltpu.SMEM((1, 1), jnp.float32),
        ],
    )(qa, ta, w)
    return out[:, 0]


# speedup vs baseline: 1.0099x; 1.0099x over previous
import math

import jax
import jax.numpy as jnp
from jax.experimental import pallas as pl
from jax.experimental.pallas import tpu as pltpu

_BW2 = 16.0
_EPS = 1e-30
_KC = 25088
_DA = 24
_LANES = 128


def _make_body(norm_const):
    def body(qa_ref, ta_ref, w_ref, out_ref, acc_ref, ws_ref):
        k = pl.program_id(0)
        nk = pl.num_programs(0)

        @pl.when(k == 0)
        def _init():
            acc_ref[...] = jnp.zeros_like(acc_ref)
            ws_ref[0, 0] = 0.0

        arg = jnp.dot(qa_ref[...], ta_ref[...],
                      preferred_element_type=jnp.float32)
        ws_ref[0, 0] += jnp.sum(w_ref[0])

        ex = jax.lax.exp2(arg.astype(jnp.bfloat16))
        parts = [ex[:, g * _LANES:(g + 1) * _LANES]
                 for g in range(_KC // _LANES)]
        while len(parts) > 1:
            nxt = [parts[i] + parts[i + 1]
                   for i in range(0, len(parts) - 1, 2)]
            if len(parts) % 2:
                nxt.append(parts[-1])
            parts = nxt
        acc_ref[...] += parts[0].astype(jnp.float32)

        @pl.when(k == nk - 1)
        def _epilogue():
            dens = jnp.sum(acc_ref[...], axis=1, keepdims=True)
            dens = dens * (norm_const / ws_ref[0, 0])
            out_ref[...] = jnp.log(jnp.maximum(dens, _EPS))

    return body


def kernel(X_query, X_train, sample_weight):
    n, d = X_query.shape
    K = X_train.shape[0]
    norm_const = (2.0 * math.pi * _BW2) ** (-d / 2.0)
    c2 = -1.0 / (2.0 * _BW2 * math.log(2.0))

    kpad = ((K + _KC - 1) // _KC) * _KC
    nk = kpad // _KC

    q_sq = jnp.sum(X_query * X_query, axis=1, keepdims=True)
    t_sq = jnp.sum(X_train * X_train, axis=1)
    lw = jnp.log2(jnp.maximum(sample_weight, 1e-38))

    ones_n = jnp.ones((n, 1), jnp.float32)
    qa = jnp.concatenate(
        [(-2.0 * c2) * X_query, c2 * q_sq, c2 * ones_n, ones_n], axis=1)
    qa = jnp.pad(qa, ((0, 0), (0, _DA - (d + 3)))).astype(jnp.bfloat16)

    ta = jnp.concatenate(
        [X_train.T, jnp.ones((1, K), jnp.float32), t_sq[None, :],
         lw[None, :]], axis=0)
    ta = jnp.pad(ta, ((0, _DA - (d + 3)), (0, 0)))
    ta = jnp.pad(ta, ((0, 0), (0, kpad - K)),
                 constant_values=0.0).at[d + 2, K:].set(-126.0)
    ta = ta.astype(jnp.bfloat16)

    w = jnp.pad(sample_weight, (0, kpad - K)).reshape(nk, 1, _KC)

    out = pl.pallas_call(
        _make_body(norm_const),
        grid=(nk,),
        in_specs=[
            pl.BlockSpec((n, _DA), lambda k: (0, 0)),
            pl.BlockSpec((_DA, _KC), lambda k: (0, k)),
            pl.BlockSpec((1, 1, _KC), lambda k: (k, 0, 0)),
        ],
        out_specs=pl.BlockSpec((n, 1), lambda k: (0, 0)),
        out_shape=jax.ShapeDtypeStruct((n, 1), jnp.float32),
        compiler_params=pltpu.CompilerParams(
            allow_input_fusion=[True, True, True]),
        scratch_shapes=[
            pltpu.VMEM((n, _LANES), jnp.float32),
            pltpu.SMEM((1, 1), jnp.float32),
        ],
    )(qa, ta, w)
    return out[:, 0]


# speedup vs baseline: 1.0199x; 1.0099x over previous
import math

import jax
import jax.numpy as jnp
from jax.experimental import pallas as pl
from jax.experimental.pallas import tpu as pltpu

_BW2 = 16.0
_EPS = 1e-30
_KC = 50176
_DA = 24
_LANES = 128


def _make_body(norm_const):
    def body(qa_ref, ta_ref, w_ref, out_ref, acc_ref, ws_ref):
        k = pl.program_id(0)
        nk = pl.num_programs(0)

        @pl.when(k == 0)
        def _init():
            acc_ref[...] = jnp.zeros_like(acc_ref)
            ws_ref[0, 0] = 0.0

        arg = jnp.dot(qa_ref[...], ta_ref[...],
                      preferred_element_type=jnp.float32)
        ws_ref[0, 0] += jnp.sum(w_ref[0])

        ex = jax.lax.exp2(arg.astype(jnp.bfloat16))
        parts = [ex[:, g * _LANES:(g + 1) * _LANES]
                 for g in range(_KC // _LANES)]
        while len(parts) > 1:
            nxt = [parts[i] + parts[i + 1]
                   for i in range(0, len(parts) - 1, 2)]
            if len(parts) % 2:
                nxt.append(parts[-1])
            parts = nxt
        acc_ref[...] += parts[0].astype(jnp.float32)

        @pl.when(k == nk - 1)
        def _epilogue():
            dens = jnp.sum(acc_ref[...], axis=1, keepdims=True)
            dens = dens * (norm_const / ws_ref[0, 0])
            out_ref[...] = jnp.log(jnp.maximum(dens, _EPS))

    return body


def kernel(X_query, X_train, sample_weight):
    n, d = X_query.shape
    K = X_train.shape[0]
    norm_const = (2.0 * math.pi * _BW2) ** (-d / 2.0)
    c2 = -1.0 / (2.0 * _BW2 * math.log(2.0))

    kpad = ((K + _KC - 1) // _KC) * _KC
    nk = kpad // _KC

    q_sq = jnp.sum(X_query * X_query, axis=1, keepdims=True)
    t_sq = jnp.sum(X_train * X_train, axis=1)
    lw = jnp.log2(jnp.maximum(sample_weight, 1e-38))

    ones_n = jnp.ones((n, 1), jnp.float32)
    qa = jnp.concatenate(
        [(-2.0 * c2) * X_query, c2 * q_sq, c2 * ones_n, ones_n], axis=1)
    qa = jnp.pad(qa, ((0, 0), (0, _DA - (d + 3)))).astype(jnp.bfloat16)

    ta = jnp.concatenate(
        [X_train.T, jnp.ones((1, K), jnp.float32), t_sq[None, :],
         lw[None, :]], axis=0)
    ta = jnp.pad(ta, ((0, _DA - (d + 3)), (0, 0)))
    ta = jnp.pad(ta, ((0, 0), (0, kpad - K)),
                 constant_values=0.0).at[d + 2, K:].set(-126.0)
    ta = ta.astype(jnp.bfloat16)

    w = jnp.pad(sample_weight, (0, kpad - K)).reshape(nk, 1, _KC)

    out = pl.pallas_call(
        _make_body(norm_const),
        grid=(nk,),
        in_specs=[
            pl.BlockSpec((n, _DA), lambda k: (0, 0)),
            pl.BlockSpec((_DA, _KC), lambda k: (0, k)),
            pl.BlockSpec((1, 1, _KC), lambda k: (k, 0, 0)),
        ],
        out_specs=pl.BlockSpec((n, 1), lambda k: (0, 0)),
        out_shape=jax.ShapeDtypeStruct((n, 1), jnp.float32),
        compiler_params=pltpu.CompilerParams(
            allow_input_fusion=[True, True, True]),
        scratch_shapes=[
            pltpu.VMEM((n, _LANES), jnp.float32),
            pltpu.SMEM((1, 1), jnp.float32),
        ],
    )(qa, ta, w)
    return out[:, 0]
